# Initial kernel scaffold; baseline (speedup 1.0000x reference)
import jax
import jax.numpy as jnp
from jax.experimental import pallas as pl


def kernel(customer_type_batch, prompt_table):
    raise NotImplementedError("write your pallas kernel here")


# speedup vs baseline: 2.5134x; 2.5134x over previous
import functools

import jax
import jax.numpy as jnp
from jax import lax
from jax.experimental import pallas as pl
from jax.experimental.pallas import tpu as pltpu
from jax.experimental.pallas import tpu_sc as plsc

NUM_TYPES = 100
BATCH = 1024
DUP = 2
NUM_HEADS = 16
LENGTH = 20
HEAD_DIM = 64
ROW = DUP * 1 * NUM_HEADS * LENGTH * HEAD_DIM

NC = 2
NS = 16
NW = NC * NS
B_PER_W = BATCH // NW


def _gather_body(table_hbm, eidx_hbm, out_hbm, eidx_v, rows_v, gsem0,
                 gsem1, psem0, psem1):
    gsem = (gsem0, gsem1)
    psem = (psem0, psem1)
    wid = lax.axis_index("s") * NC + lax.axis_index("c")
    base = wid * B_PER_W

    pltpu.sync_copy(eidx_hbm.at[pl.ds(base * 8, B_PER_W * 8)], eidx_v)

    def start_gather(j, t):
        pltpu.make_async_copy(
            table_hbm.at[eidx_v.at[pl.ds(8 * j, 1)]], rows_v.at[t], gsem[t]
        ).start()

    def wait_gather(j, t):
        pltpu.make_async_copy(
            table_hbm.at[eidx_v.at[pl.ds(8 * j, 1)]], rows_v.at[t], gsem[t]
        ).wait()

    start_gather(0, 0)
    start_gather(1, 1)

    def pair(i, carry):
        for t in range(2):
            j = 2 * i + t
            wait_gather(j, t)
            put = pltpu.make_async_copy(
                rows_v.at[t], out_hbm.at[pl.ds(base + j, 1)], psem[t]
            )
            put.start()
            put.wait()
            start_gather(j + 2, t)
        return carry

    lax.fori_loop(0, B_PER_W // 2 - 1, pair, 0)

    for t in range(2):
        j = B_PER_W - 2 + t
        wait_gather(j, t)
        pltpu.make_async_copy(
            rows_v.at[t], out_hbm.at[pl.ds(base + j, 1)], psem[t]
        ).start()
    for t in range(2):
        j = B_PER_W - 2 + t
        pltpu.make_async_copy(
            rows_v.at[t], out_hbm.at[pl.ds(base + j, 1)], psem[t]
        ).wait()


@functools.partial(jax.jit, static_argnames=())
def _gather(table, idx):
    mesh = plsc.VectorSubcoreMesh(core_axis_name="c", subcore_axis_name="s")
    return pl.kernel(
        _gather_body,
        out_type=jax.ShapeDtypeStruct((BATCH, ROW), jnp.float32),
        mesh=mesh,
        scratch_types=[
            pltpu.VMEM((B_PER_W * 8,), jnp.int32),
            pltpu.VMEM((2, 1, ROW), jnp.float32),
            pltpu.SemaphoreType.DMA,
            pltpu.SemaphoreType.DMA,
            pltpu.SemaphoreType.DMA,
            pltpu.SemaphoreType.DMA,
        ],
    )(table, idx)


def kernel(customer_type_batch, prompt_table):
    idx = jnp.repeat(customer_type_batch.astype(jnp.int32), 8)
    table = prompt_table.reshape(NUM_TYPES, ROW)
    out = _gather(table, idx)
    return out.reshape(BATCH, DUP, 1, NUM_HEADS, LENGTH, HEAD_DIM)
